# baseline (speedup 1.0000x reference)
import functools

import jax
import jax.numpy as jnp
from jax import lax
from jax.experimental import pallas as pl
from jax.experimental.pallas import tpu as pltpu
from jax.experimental.pallas import tpu_sc as plsc

B = 8
D = 256
M = 4
N = 2048
NV = N // 16
NP = NV // 2
NLEV = 7
NROWS = B * D * M
NW = 32
ROWS_PER_TILE = NROWS // NW
D_PER_TILE = D // NW


def _sort16a(v):
  return plsc.sort_key_val(v, v)[0]


def _sort16d(v):
  return plsc.sort_key_val(v, v, descending=True)[0]


def _sort_row_inplace(row_v):
  for j in range(4):
    n = 2 << j
    nblocks = NV // n

    @plsc.parallel_loop(0, nblocks, unroll=max(1, 8 >> j))
    def small_merge(bi, j=j, n=n):
      base = bi * (n * 16)
      v = [row_v[pl.ds(base + q * 16, 16)] for q in range(n)]
      for t in range(n // 2):
        a = _sort16a(v[t])
        b = _sort16d(v[n - 1 - t])
        v[t] = jnp.minimum(a, b)
        v[n - 1 - t] = jnp.flip(jnp.maximum(a, b), axis=0)
      for e in range(j - 1, -1, -1):
        d = 1 << e
        for g in range(n // (2 * d)):
          for t in range(d):
            i0 = g * 2 * d + t
            a = v[i0]
            b = v[i0 + d]
            v[i0] = jnp.minimum(a, b)
            v[i0 + d] = jnp.maximum(a, b)
      for q in range(n):
        row_v[pl.ds(base + q * 16, 16)] = v[q]

  for j in range(4, NLEV):
    m = 1 << j

    @plsc.parallel_loop(0, NP, unroll=8)
    def stage1(k, j=j, m=m):
      p = k // m
      t = k % m
      lo = p * (2 * m)
      ia = (lo + t) * 16
      ib = (lo + 2 * m - 1 - t) * 16
      a = _sort16a(row_v[pl.ds(ia, 16)])
      b = _sort16d(row_v[pl.ds(ib, 16)])
      row_v[pl.ds(ia, 16)] = jnp.minimum(a, b)
      row_v[pl.ds(ib, 16)] = jnp.flip(jnp.maximum(a, b), axis=0)

    for e in range(j - 1, 3, -1):
      d = 1 << e

      @plsc.parallel_loop(0, NP, unroll=8)
      def xstage(k, d=d, e=e):
        g = k // d
        t = k % d
        i0 = (g * 2 * d + t) * 16
        a = row_v[pl.ds(i0, 16)]
        b = row_v[pl.ds(i0 + d * 16, 16)]
        row_v[pl.ds(i0, 16)] = jnp.minimum(a, b)
        row_v[pl.ds(i0 + d * 16, 16)] = jnp.maximum(a, b)

    @plsc.parallel_loop(0, NV // 16, unroll=2)
    def xor_tail(bi):
      base = bi * 256
      v = [row_v[pl.ds(base + q * 16, 16)] for q in range(16)]
      for e in (3, 2, 1, 0):
        d = 1 << e
        for g in range(16 // (2 * d)):
          for t in range(d):
            i0 = g * 2 * d + t
            a = v[i0]
            b = v[i0 + d]
            v[i0] = jnp.minimum(a, b)
            v[i0 + d] = jnp.maximum(a, b)
      for q in range(16):
        row_v[pl.ds(base + q * 16, 16)] = v[q]


def _fspool_body(x_hbm, w_hbm, out_hbm, row_a, row_b, wrow_v, res_v,
                 sem_a, sem_b):
  wid = lax.axis_index("s") * 2 + lax.axis_index("c")

  def row_global_idx(t):
    d_idx = wid * D_PER_TILE + t // (B * M)
    r = t % (B * M)
    return (r // M) * (D * M) + d_idx * M + (r % M)

  def process(t, row_v, sem_p, row_q, sem_q, res_vreg):
    pltpu.make_async_copy(x_hbm.at[0], row_v, sem_p).wait()

    @pl.when(t < ROWS_PER_TILE - 1)
    def _():
      pltpu.async_copy(x_hbm.at[row_global_idx(t + 1)], row_q, sem_q)

    @pl.when(t % (B * M) == 0)
    def _():
      pltpu.sync_copy(w_hbm.at[wid * D_PER_TILE + t // (B * M)], wrow_v)

    _sort_row_inplace(row_v)

    zero = jnp.zeros((16,), jnp.float32)

    @plsc.parallel_loop(0, NV, step=4, carry=(zero, zero, zero, zero))
    def wsum(i, accs):
      a0, a1, a2, a3 = accs
      s0 = _sort16a(row_v[pl.ds(i * 16, 16)])
      s1 = _sort16a(row_v[pl.ds((i + 1) * 16, 16)])
      s2 = _sort16a(row_v[pl.ds((i + 2) * 16, 16)])
      s3 = _sort16a(row_v[pl.ds((i + 3) * 16, 16)])
      return (a0 + s0 * wrow_v[pl.ds(i * 16, 16)],
              a1 + s1 * wrow_v[pl.ds((i + 1) * 16, 16)],
              a2 + s2 * wrow_v[pl.ds((i + 2) * 16, 16)],
              a3 + s3 * wrow_v[pl.ds((i + 3) * 16, 16)])

    a0, a1, a2, a3 = wsum
    total = jnp.sum((a0 + a1) + (a2 + a3))
    res_vreg = jnp.where(lax.iota(jnp.int32, 16) == t % 16, total, res_vreg)

    @pl.when(t % 16 == 15)
    def _():
      res_v[pl.ds((t // 16) * 16, 16)] = res_vreg

    return res_vreg

  pltpu.async_copy(x_hbm.at[row_global_idx(0)], row_a, sem_a)

  def row_pair(t2, res_vreg):
    t = 2 * t2
    res_vreg = process(t, row_a, sem_a, row_b, sem_b, res_vreg)
    res_vreg = process(t + 1, row_b, sem_b, row_a, sem_a, res_vreg)
    return res_vreg

  lax.fori_loop(0, ROWS_PER_TILE // 2, row_pair, jnp.zeros((16,), jnp.float32))
  pltpu.sync_copy(res_v, out_hbm.at[pl.ds(wid * ROWS_PER_TILE,
                                          ROWS_PER_TILE)])


@jax.jit
def kernel(x, weight):
  xr = x.reshape(NROWS, N)
  mesh = plsc.VectorSubcoreMesh(core_axis_name="c", subcore_axis_name="s",
                                num_cores=2, num_subcores=16)
  fs = pl.kernel(
      _fspool_body,
      out_type=jax.ShapeDtypeStruct((NROWS,), jnp.float32),
      mesh=mesh,
      scratch_types=[
          pltpu.VMEM((N,), jnp.float32),
          pltpu.VMEM((N,), jnp.float32),
          pltpu.VMEM((N,), jnp.float32),
          pltpu.VMEM((ROWS_PER_TILE,), jnp.float32),
          pltpu.SemaphoreType.DMA,
          pltpu.SemaphoreType.DMA,
      ],
      compiler_params=pltpu.CompilerParams(needs_layout_passes=False),
  )
  out = fs(xr, weight)
  return out.reshape(D, B, M).transpose(1, 0, 2)


# speedup vs baseline: 1.0844x; 1.0844x over previous
import functools

import jax
import jax.numpy as jnp
from jax import lax
from jax.experimental import pallas as pl
from jax.experimental.pallas import tpu as pltpu
from jax.experimental.pallas import tpu_sc as plsc

B = 8
D = 256
M = 4
N = 2048
NV = N // 16
NP = NV // 2
NLEV = 7
NROWS = B * D * M
NW = 32
ROWS_PER_TILE = NROWS // NW
D_PER_TILE = D // NW


def _sort16a(v):
  return plsc.sort_key_val(v, v)[0]


def _sort16d(v):
  return plsc.sort_key_val(v, v, descending=True)[0]


def _sort_row_inplace(row_v):
  for j in range(4):
    n = 2 << j
    nblocks = NV // n

    @plsc.parallel_loop(0, nblocks, unroll=max(1, 8 >> j))
    def small_merge(bi, j=j, n=n):
      base = bi * (n * 16)
      v = [row_v[pl.ds(base + q * 16, 16)] for q in range(n)]
      for t in range(n // 2):
        a = _sort16a(v[t])
        b = _sort16d(v[n - 1 - t])
        v[t] = jnp.minimum(a, b)
        v[n - 1 - t] = jnp.flip(jnp.maximum(a, b), axis=0)
      for e in range(j - 1, -1, -1):
        d = 1 << e
        for g in range(n // (2 * d)):
          for t in range(d):
            i0 = g * 2 * d + t
            a = v[i0]
            b = v[i0 + d]
            v[i0] = jnp.minimum(a, b)
            v[i0 + d] = jnp.maximum(a, b)
      for q in range(n):
        row_v[pl.ds(base + q * 16, 16)] = v[q]

  for j in range(4, NLEV):
    m = 1 << j

    @plsc.parallel_loop(0, NP, unroll=8)
    def stage1(k, j=j, m=m):
      p = k // m
      t = k % m
      lo = p * (2 * m)
      ia = (lo + t) * 16
      ib = (lo + 2 * m - 1 - t) * 16
      a = _sort16a(row_v[pl.ds(ia, 16)])
      b = _sort16d(row_v[pl.ds(ib, 16)])
      row_v[pl.ds(ia, 16)] = jnp.minimum(a, b)
      row_v[pl.ds(ib, 16)] = jnp.flip(jnp.maximum(a, b), axis=0)

    for e in range(j - 1, 3, -1):
      d = 1 << e

      @plsc.parallel_loop(0, NP, unroll=8)
      def xstage(k, d=d, e=e):
        g = k // d
        t = k % d
        i0 = (g * 2 * d + t) * 16
        a = row_v[pl.ds(i0, 16)]
        b = row_v[pl.ds(i0 + d * 16, 16)]
        row_v[pl.ds(i0, 16)] = jnp.minimum(a, b)
        row_v[pl.ds(i0 + d * 16, 16)] = jnp.maximum(a, b)

    if j == NLEV - 1:
      continue

    @plsc.parallel_loop(0, NV // 16, unroll=1)
    def xor_tail(bi):
      base = bi * 256
      v = [row_v[pl.ds(base + q * 16, 16)] for q in range(16)]
      for e in (3, 2, 1, 0):
        d = 1 << e
        for g in range(16 // (2 * d)):
          for t in range(d):
            i0 = g * 2 * d + t
            a = v[i0]
            b = v[i0 + d]
            v[i0] = jnp.minimum(a, b)
            v[i0 + d] = jnp.maximum(a, b)
      for q in range(16):
        row_v[pl.ds(base + q * 16, 16)] = v[q]


def _tail_and_wsum(row_v, wrow_v):
  zero = jnp.zeros((16,), jnp.float32)

  @plsc.parallel_loop(0, NV // 16, carry=(zero, zero, zero, zero))
  def tail_wsum(bi, accs):
    a0, a1, a2, a3 = accs
    base = bi * 256
    v = [row_v[pl.ds(base + q * 16, 16)] for q in range(16)]
    for e in (3, 2, 1, 0):
      d = 1 << e
      for g in range(16 // (2 * d)):
        for t in range(d):
          i0 = g * 2 * d + t
          a = v[i0]
          b = v[i0 + d]
          v[i0] = jnp.minimum(a, b)
          v[i0 + d] = jnp.maximum(a, b)
    ws = [wrow_v[pl.ds(base + q * 16, 16)] for q in range(16)]
    for q in range(0, 16, 4):
      a0 = a0 + _sort16a(v[q]) * ws[q]
      a1 = a1 + _sort16a(v[q + 1]) * ws[q + 1]
      a2 = a2 + _sort16a(v[q + 2]) * ws[q + 2]
      a3 = a3 + _sort16a(v[q + 3]) * ws[q + 3]
    return (a0, a1, a2, a3)

  a0, a1, a2, a3 = tail_wsum
  return jnp.sum((a0 + a1) + (a2 + a3))


def _fspool_body(x_hbm, w_hbm, out_hbm, row_a, row_b, wrow_v, res_v,
                 sem_a, sem_b):
  wid = lax.axis_index("s") * 2 + lax.axis_index("c")

  def row_global_idx(t):
    d_idx = wid * D_PER_TILE + t // (B * M)
    r = t % (B * M)
    return (r // M) * (D * M) + d_idx * M + (r % M)

  def process(t, row_v, sem_p, row_q, sem_q, res_vreg):
    pltpu.make_async_copy(x_hbm.at[0], row_v, sem_p).wait()

    @pl.when(t < ROWS_PER_TILE - 1)
    def _():
      pltpu.async_copy(x_hbm.at[row_global_idx(t + 1)], row_q, sem_q)

    @pl.when(t % (B * M) == 0)
    def _():
      pltpu.sync_copy(w_hbm.at[wid * D_PER_TILE + t // (B * M)], wrow_v)

    _sort_row_inplace(row_v)
    total = _tail_and_wsum(row_v, wrow_v)
    res_vreg = jnp.where(lax.iota(jnp.int32, 16) == t % 16, total, res_vreg)

    @pl.when(t % 16 == 15)
    def _():
      res_v[pl.ds((t // 16) * 16, 16)] = res_vreg

    return res_vreg

  pltpu.async_copy(x_hbm.at[row_global_idx(0)], row_a, sem_a)

  def row_pair(t2, res_vreg):
    t = 2 * t2
    res_vreg = process(t, row_a, sem_a, row_b, sem_b, res_vreg)
    res_vreg = process(t + 1, row_b, sem_b, row_a, sem_a, res_vreg)
    return res_vreg

  lax.fori_loop(0, ROWS_PER_TILE // 2, row_pair, jnp.zeros((16,), jnp.float32))
  pltpu.sync_copy(res_v, out_hbm.at[pl.ds(wid * ROWS_PER_TILE,
                                          ROWS_PER_TILE)])


@jax.jit
def kernel(x, weight):
  xr = x.reshape(NROWS, N)
  mesh = plsc.VectorSubcoreMesh(core_axis_name="c", subcore_axis_name="s",
                                num_cores=2, num_subcores=16)
  fs = pl.kernel(
      _fspool_body,
      out_type=jax.ShapeDtypeStruct((NROWS,), jnp.float32),
      mesh=mesh,
      scratch_types=[
          pltpu.VMEM((N,), jnp.float32),
          pltpu.VMEM((N,), jnp.float32),
          pltpu.VMEM((N,), jnp.float32),
          pltpu.VMEM((ROWS_PER_TILE,), jnp.float32),
          pltpu.SemaphoreType.DMA,
          pltpu.SemaphoreType.DMA,
      ],
      compiler_params=pltpu.CompilerParams(needs_layout_passes=False),
  )
  out = fs(xr, weight)
  return out.reshape(D, B, M).transpose(1, 0, 2)
